# baseline (speedup 1.0000x reference)
import functools

import jax
import jax.numpy as jnp
from jax import lax
from jax.experimental import pallas as pl
from jax.experimental.pallas import tpu as pltpu
from jax.experimental.pallas import tpu_sc as plsc

N = 10000
E = 320000
D = 128
NC = 2
NS = 16
NW = NC * NS
EPW = E // NW
CHUNK = 125
NCHUNK = EPW // CHUNK
NBUF = 2
SUPER = 16
NSUPER = NCHUNK // SUPER
NPAD = 10240
STRIPE = NPAD // NS
ONESLEN = -(-CHUNK // 16) * 16

_MESH = functools.partial(
    plsc.VectorSubcoreMesh, core_axis_name="c", subcore_axis_name="s")


def _deg_body(row_hbm, zeros_hbm, out_hbm, idx_v, ones_v, deg_sh):
    cid = lax.axis_index("c")
    sid = lax.axis_index("s")
    wid = sid * NC + cid
    pltpu.sync_copy(zeros_hbm, deg_sh.at[pl.ds(sid * STRIPE, STRIPE)])
    for v in range(ONESLEN // 16):
        ones_v[pl.ds(v * 16, 16)] = jnp.ones((16,), jnp.float32)
    plsc.subcore_barrier()

    for s in range(NSUPER):
        pltpu.sync_copy(row_hbm.at[wid, s], idx_v)

        def body(j, carry):
            pltpu.sync_copy(ones_v.at[pl.ds(0, CHUNK)],
                            deg_sh.at[idx_v.at[j]], add=True)
            return carry

        lax.fori_loop(0, SUPER, body, 0)
    plsc.subcore_barrier()
    pltpu.sync_copy(deg_sh.at[pl.ds(sid * STRIPE, STRIPE)],
                    out_hbm.at[cid, 0, pl.ds(sid * STRIPE, STRIPE)])


_deg_kernel = functools.partial(
    pl.kernel,
    out_type=jax.ShapeDtypeStruct((NC, 1, NPAD), jnp.float32),
    mesh=_MESH(),
    scratch_types=[
        pltpu.VMEM((SUPER, CHUNK), jnp.int32),
        pltpu.VMEM((ONESLEN,), jnp.float32),
        pltpu.VMEM_SHARED((NPAD,), jnp.float32),
    ],
)(_deg_body)


def _spmm_body(y_hbm, col_hbm, row_hbm, zeros_hbm, out_hbm,
               idx_c, idx_r, buf0, buf1, acc_sh, gsem0, gsem1, ssem0, ssem1):
    bufs = (buf0, buf1)
    gsem = (gsem0, gsem1)
    ssem = (ssem0, ssem1)
    cid = lax.axis_index("c")
    sid = lax.axis_index("s")
    wid = sid * NC + cid
    pltpu.sync_copy(zeros_hbm, acc_sh.at[pl.ds(sid * STRIPE, STRIPE)])
    plsc.subcore_barrier()

    for s in range(NSUPER):
        pltpu.sync_copy(col_hbm.at[wid, s], idx_c)
        pltpu.sync_copy(row_hbm.at[wid, s], idx_r)
        pltpu.async_copy(y_hbm.at[idx_c.at[0]], bufs[0], gsem[0])

        def group(g, carry):
            for b in range(NBUF):
                j = g * NBUF + b
                pltpu.make_async_copy(y_hbm.at[idx_c.at[j]], bufs[b],
                                      gsem[b]).wait()

                @pl.when(j >= 1)
                def _():
                    pltpu.make_async_copy(bufs[1 - b],
                                          acc_sh.at[idx_r.at[j - 1]],
                                          ssem[1 - b]).wait()

                pltpu.async_copy(bufs[b], acc_sh.at[idx_r.at[j]],
                                 ssem[b], add=True)

                @pl.when(j + 1 < SUPER)
                def _():
                    pltpu.async_copy(y_hbm.at[idx_c.at[j + 1]],
                                     bufs[1 - b], gsem[1 - b])
            return carry

        lax.fori_loop(0, SUPER // NBUF, group, 0)
        pltpu.make_async_copy(bufs[1], acc_sh.at[idx_r.at[SUPER - 1]],
                              ssem[1]).wait()
    plsc.subcore_barrier()
    pltpu.sync_copy(acc_sh.at[pl.ds(sid * STRIPE, STRIPE)],
                    out_hbm.at[cid, pl.ds(sid * STRIPE, STRIPE)])


_spmm_kernel = functools.partial(
    pl.kernel,
    out_type=jax.ShapeDtypeStruct((NC, NPAD, D), jnp.float32),
    mesh=_MESH(),
    scratch_types=[
        pltpu.VMEM((SUPER, CHUNK), jnp.int32),
        pltpu.VMEM((SUPER, CHUNK), jnp.int32),
        pltpu.VMEM((CHUNK, D), jnp.float32),
        pltpu.VMEM((CHUNK, D), jnp.float32),
        pltpu.VMEM_SHARED((NPAD, D), jnp.float32),
        pltpu.SemaphoreType.DMA,
        pltpu.SemaphoreType.DMA,
        pltpu.SemaphoreType.DMA,
        pltpu.SemaphoreType.DMA,
    ],
)(_spmm_body)


BR = 2000


def _scale_body(p0_ref, p1_ref, x_ref, y_ref, dis_ref):
    deg = p0_ref[...] + p1_ref[...]
    dis = jnp.where(deg > 0.0, lax.rsqrt(deg), 0.0)
    dis_ref[...] = dis
    y_ref[...] = dis * x_ref[...]


def _tc_scale(p0, p1, x):
    return pl.pallas_call(
        _scale_body,
        grid=(N // BR,),
        in_specs=[
            pl.BlockSpec((BR, 1), lambda i: (i, 0)),
            pl.BlockSpec((BR, 1), lambda i: (i, 0)),
            pl.BlockSpec((BR, D), lambda i: (i, 0)),
        ],
        out_specs=[
            pl.BlockSpec((BR, D), lambda i: (i, 0)),
            pl.BlockSpec((BR, 1), lambda i: (i, 0)),
        ],
        out_shape=[
            jax.ShapeDtypeStruct((N, D), jnp.float32),
            jax.ShapeDtypeStruct((N, 1), jnp.float32),
        ],
    )(p0, p1, x)


def _mid_body(dis_ref, q0_ref, q1_ref, tx1_ref, y2_ref):
    dis = dis_ref[...]
    tx1 = -dis * (q0_ref[0] + q1_ref[0])
    tx1_ref[...] = tx1
    y2_ref[...] = dis * tx1


def _tc_mid(dis, agg):
    return pl.pallas_call(
        _mid_body,
        grid=(N // BR,),
        in_specs=[
            pl.BlockSpec((BR, 1), lambda i: (i, 0)),
            pl.BlockSpec((1, BR, D), lambda i: (0, i, 0)),
            pl.BlockSpec((1, BR, D), lambda i: (1, i, 0)),
        ],
        out_specs=[
            pl.BlockSpec((BR, D), lambda i: (i, 0)),
            pl.BlockSpec((BR, D), lambda i: (i, 0)),
        ],
        out_shape=[
            jax.ShapeDtypeStruct((N, D), jnp.float32),
            jax.ShapeDtypeStruct((N, D), jnp.float32),
        ],
    )(dis, agg, agg)


def _final_body(x_ref, tx1_ref, dis_ref, q0_ref, q1_ref,
                wz_ref, wh_ref, bz_ref, bh_ref, h_ref):
    x = x_ref[...]
    tx1 = tx1_ref[...]
    tx2 = -2.0 * dis_ref[...] * (q0_ref[0] + q1_ref[0]) - x
    dot = functools.partial(
        jnp.dot, precision=lax.Precision.HIGHEST,
        preferred_element_type=jnp.float32)
    sz = (dot(x, wz_ref[0]) + dot(tx1, wz_ref[1]) + dot(tx2, wz_ref[2])
          + bz_ref[...])
    sh = (dot(x, wh_ref[0]) + dot(tx1, wh_ref[1]) + dot(tx2, wh_ref[2])
          + bh_ref[...])
    h_ref[...] = (1.0 - jax.nn.sigmoid(sz)) * jnp.tanh(sh)


def _tc_final(x, tx1, dis, agg, wz, wh, bz, bh):
    return pl.pallas_call(
        _final_body,
        grid=(N // BR,),
        in_specs=[
            pl.BlockSpec((BR, D), lambda i: (i, 0)),
            pl.BlockSpec((BR, D), lambda i: (i, 0)),
            pl.BlockSpec((BR, 1), lambda i: (i, 0)),
            pl.BlockSpec((1, BR, D), lambda i: (0, i, 0)),
            pl.BlockSpec((1, BR, D), lambda i: (1, i, 0)),
            pl.BlockSpec((3, D, D), lambda i: (0, 0, 0)),
            pl.BlockSpec((3, D, D), lambda i: (0, 0, 0)),
            pl.BlockSpec((1, D), lambda i: (0, 0)),
            pl.BlockSpec((1, D), lambda i: (0, 0)),
        ],
        out_specs=pl.BlockSpec((BR, D), lambda i: (i, 0)),
        out_shape=jax.ShapeDtypeStruct((N, D), jnp.float32),
    )(x, tx1, dis, agg, agg, wz, wh, bz, bh)


def kernel(X, edge_index, W_xz, b_xz, W_hz, b_hz, W_xr, b_xr, W_hr, b_hr,
           W_xh, b_xh, W_hh, b_hh):
    row4d = edge_index[0].reshape(NW, NSUPER, SUPER, CHUNK)
    col4d = edge_index[1].reshape(NW, NSUPER, SUPER, CHUNK)
    z_row = jnp.zeros((STRIPE,), jnp.float32)
    z_acc = jnp.zeros((STRIPE, D), jnp.float32)

    deg_p = _deg_kernel(row4d, z_row)
    p0 = deg_p[0, 0, :N, None]
    p1 = deg_p[1, 0, :N, None]
    y1, dis = _tc_scale(p0, p1, X)

    agg1 = _spmm_kernel(y1, col4d, row4d, z_acc)
    tx1, y2 = _tc_mid(dis, agg1)

    agg2 = _spmm_kernel(y2, col4d, row4d, z_acc)
    bz = (b_xz + b_hz)[None, :]
    bh = (b_xh + b_hh)[None, :]
    return _tc_final(X, tx1, dis, agg2, W_xz, W_xh, bz, bh)


# speedup vs baseline: 1.1161x; 1.1161x over previous
import functools

import jax
import jax.numpy as jnp
from jax import lax
from jax.experimental import pallas as pl
from jax.experimental.pallas import tpu as pltpu
from jax.experimental.pallas import tpu_sc as plsc

N = 10000
E = 320000
D = 128
NC = 2
NS = 16
NW = NC * NS
EPW = E // NW
CHUNK = 125
NCHUNK = EPW // CHUNK
NBUF = 2
SUPER = 16
NSUPER = NCHUNK // SUPER
NPAD = 10240
STRIPE = NPAD // NS
ONESLEN = -(-CHUNK // 16) * 16

_MESH = functools.partial(
    plsc.VectorSubcoreMesh, core_axis_name="c", subcore_axis_name="s")


def _deg_body(row_hbm, zeros_hbm, out_hbm, idx_v, ones_v, deg_sh):
    cid = lax.axis_index("c")
    sid = lax.axis_index("s")
    wid = sid * NC + cid
    pltpu.sync_copy(zeros_hbm, deg_sh.at[pl.ds(sid * STRIPE, STRIPE)])
    for v in range(ONESLEN // 16):
        ones_v[pl.ds(v * 16, 16)] = jnp.ones((16,), jnp.float32)
    plsc.subcore_barrier()

    for s in range(NSUPER):
        pltpu.sync_copy(row_hbm.at[wid, s], idx_v)

        def body(j, carry):
            pltpu.sync_copy(ones_v.at[pl.ds(0, CHUNK)],
                            deg_sh.at[idx_v.at[j]], add=True)
            return carry

        lax.fori_loop(0, SUPER, body, 0)
    plsc.subcore_barrier()
    pltpu.sync_copy(deg_sh.at[pl.ds(sid * STRIPE, STRIPE)],
                    out_hbm.at[cid, 0, pl.ds(sid * STRIPE, STRIPE)])


_deg_kernel = functools.partial(
    pl.kernel,
    out_type=jax.ShapeDtypeStruct((NC, 1, NPAD), jnp.float32),
    mesh=_MESH(),
    scratch_types=[
        pltpu.VMEM((SUPER, CHUNK), jnp.int32),
        pltpu.VMEM((ONESLEN,), jnp.float32),
        pltpu.VMEM_SHARED((NPAD,), jnp.float32),
    ],
)(_deg_body)


def _spmm_body(y_hbm, col_hbm, row_hbm, zeros_hbm, out_hbm,
               idx_c, idx_r, buf0, buf1, acc_sh, sem0, sem1):
    bufs = (buf0, buf1)
    sems = (sem0, sem1)
    cid = lax.axis_index("c")
    sid = lax.axis_index("s")
    wid = sid * NC + cid
    pltpu.sync_copy(zeros_hbm, acc_sh.at[pl.ds(sid * STRIPE, STRIPE)])
    plsc.subcore_barrier()

    for s in range(NSUPER):
        pltpu.sync_copy(col_hbm.at[wid, s], idx_c)
        pltpu.sync_copy(row_hbm.at[wid, s], idx_r)
        pltpu.async_copy(y_hbm.at[idx_c.at[0]], bufs[0], sems[0])

        def group(g, carry):
            for b in range(NBUF):
                j = g * NBUF + b
                nxt = j + 1
                nb = (b + 1) % NBUF

                @pl.when(nxt < SUPER)
                def _():
                    pltpu.async_copy(y_hbm.at[idx_c.at[nxt]], bufs[nb],
                                     sems[nb])

                pltpu.make_async_copy(y_hbm.at[idx_c.at[j]], bufs[b],
                                      sems[b]).wait()
                pltpu.sync_copy(bufs[b], acc_sh.at[idx_r.at[j]], add=True)
            return carry

        lax.fori_loop(0, SUPER // NBUF, group, 0)
    plsc.subcore_barrier()
    pltpu.sync_copy(acc_sh.at[pl.ds(sid * STRIPE, STRIPE)],
                    out_hbm.at[cid, pl.ds(sid * STRIPE, STRIPE)])


_spmm_kernel = functools.partial(
    pl.kernel,
    out_type=jax.ShapeDtypeStruct((NC, NPAD, D), jnp.float32),
    mesh=_MESH(),
    scratch_types=[
        pltpu.VMEM((SUPER, CHUNK), jnp.int32),
        pltpu.VMEM((SUPER, CHUNK), jnp.int32),
        pltpu.VMEM((CHUNK, D), jnp.float32),
        pltpu.VMEM((CHUNK, D), jnp.float32),
        pltpu.VMEM_SHARED((NPAD, D), jnp.float32),
        pltpu.SemaphoreType.DMA,
        pltpu.SemaphoreType.DMA,
    ],
)(_spmm_body)


BR = 2000


def _scale_body(p0_ref, p1_ref, x_ref, y_ref, dis_ref):
    deg = p0_ref[...] + p1_ref[...]
    dis = jnp.where(deg > 0.0, lax.rsqrt(deg), 0.0)
    dis_ref[...] = dis
    y_ref[...] = dis * x_ref[...]


def _tc_scale(p0, p1, x):
    return pl.pallas_call(
        _scale_body,
        grid=(N // BR,),
        in_specs=[
            pl.BlockSpec((BR, 1), lambda i: (i, 0)),
            pl.BlockSpec((BR, 1), lambda i: (i, 0)),
            pl.BlockSpec((BR, D), lambda i: (i, 0)),
        ],
        out_specs=[
            pl.BlockSpec((BR, D), lambda i: (i, 0)),
            pl.BlockSpec((BR, 1), lambda i: (i, 0)),
        ],
        out_shape=[
            jax.ShapeDtypeStruct((N, D), jnp.float32),
            jax.ShapeDtypeStruct((N, 1), jnp.float32),
        ],
    )(p0, p1, x)


def _mid_body(dis_ref, q0_ref, q1_ref, tx1_ref, y2_ref):
    dis = dis_ref[...]
    tx1 = -dis * (q0_ref[0] + q1_ref[0])
    tx1_ref[...] = tx1
    y2_ref[...] = dis * tx1


def _tc_mid(dis, agg):
    return pl.pallas_call(
        _mid_body,
        grid=(N // BR,),
        in_specs=[
            pl.BlockSpec((BR, 1), lambda i: (i, 0)),
            pl.BlockSpec((1, BR, D), lambda i: (0, i, 0)),
            pl.BlockSpec((1, BR, D), lambda i: (1, i, 0)),
        ],
        out_specs=[
            pl.BlockSpec((BR, D), lambda i: (i, 0)),
            pl.BlockSpec((BR, D), lambda i: (i, 0)),
        ],
        out_shape=[
            jax.ShapeDtypeStruct((N, D), jnp.float32),
            jax.ShapeDtypeStruct((N, D), jnp.float32),
        ],
    )(dis, agg, agg)


def _final_body(x_ref, tx1_ref, dis_ref, q0_ref, q1_ref,
                wz_ref, wh_ref, bz_ref, bh_ref, h_ref):
    x = x_ref[...]
    tx1 = tx1_ref[...]
    tx2 = -2.0 * dis_ref[...] * (q0_ref[0] + q1_ref[0]) - x
    dot = functools.partial(
        jnp.dot, precision=lax.Precision.HIGHEST,
        preferred_element_type=jnp.float32)
    sz = (dot(x, wz_ref[0]) + dot(tx1, wz_ref[1]) + dot(tx2, wz_ref[2])
          + bz_ref[...])
    sh = (dot(x, wh_ref[0]) + dot(tx1, wh_ref[1]) + dot(tx2, wh_ref[2])
          + bh_ref[...])
    h_ref[...] = (1.0 - jax.nn.sigmoid(sz)) * jnp.tanh(sh)


def _tc_final(x, tx1, dis, agg, wz, wh, bz, bh):
    return pl.pallas_call(
        _final_body,
        grid=(N // BR,),
        in_specs=[
            pl.BlockSpec((BR, D), lambda i: (i, 0)),
            pl.BlockSpec((BR, D), lambda i: (i, 0)),
            pl.BlockSpec((BR, 1), lambda i: (i, 0)),
            pl.BlockSpec((1, BR, D), lambda i: (0, i, 0)),
            pl.BlockSpec((1, BR, D), lambda i: (1, i, 0)),
            pl.BlockSpec((3, D, D), lambda i: (0, 0, 0)),
            pl.BlockSpec((3, D, D), lambda i: (0, 0, 0)),
            pl.BlockSpec((1, D), lambda i: (0, 0)),
            pl.BlockSpec((1, D), lambda i: (0, 0)),
        ],
        out_specs=pl.BlockSpec((BR, D), lambda i: (i, 0)),
        out_shape=jax.ShapeDtypeStruct((N, D), jnp.float32),
    )(x, tx1, dis, agg, agg, wz, wh, bz, bh)


def kernel(X, edge_index, W_xz, b_xz, W_hz, b_hz, W_xr, b_xr, W_hr, b_hr,
           W_xh, b_xh, W_hh, b_hh):
    row4d = edge_index[0].reshape(NW, NSUPER, SUPER, CHUNK)
    col4d = edge_index[1].reshape(NW, NSUPER, SUPER, CHUNK)
    z_row = jnp.zeros((STRIPE,), jnp.float32)
    z_acc = jnp.zeros((STRIPE, D), jnp.float32)

    deg_p = _deg_kernel(row4d, z_row)
    p0 = deg_p[0, 0, :N, None]
    p1 = deg_p[1, 0, :N, None]
    y1, dis = _tc_scale(p0, p1, X)

    agg1 = _spmm_kernel(y1, col4d, row4d, z_acc)
    tx1, y2 = _tc_mid(dis, agg1)

    agg2 = _spmm_kernel(y2, col4d, row4d, z_acc)
    bz = (b_xz + b_hz)[None, :]
    bh = (b_xh + b_hh)[None, :]
    return _tc_final(X, tx1, dis, agg2, W_xz, W_xh, bz, bh)


# speedup vs baseline: 1.1187x; 1.0023x over previous
import functools

import jax
import jax.numpy as jnp
from jax import lax
from jax.experimental import pallas as pl
from jax.experimental.pallas import tpu as pltpu
from jax.experimental.pallas import tpu_sc as plsc

N = 10000
E = 320000
D = 128
NC = 2
NS = 16
NW = NC * NS
EPW = E // NW
CHUNK = 125
NCHUNK = EPW // CHUNK
NBUF = 2
SUPER = 16
NSUPER = NCHUNK // SUPER
NPAD = 10240
STRIPE = NPAD // NS
DCHUNK = 80
DNCHUNK = EPW // DCHUNK

_MESH = functools.partial(
    plsc.VectorSubcoreMesh, core_axis_name="c", subcore_axis_name="s")


def _deg_body(row_hbm, zeros_hbm, out_hbm, idx_v, ones_v, deg_sh):
    cid = lax.axis_index("c")
    sid = lax.axis_index("s")
    wid = sid * NC + cid
    pltpu.sync_copy(row_hbm.at[wid], idx_v)
    pltpu.sync_copy(zeros_hbm, deg_sh.at[pl.ds(sid * STRIPE, STRIPE)])
    for v in range(DCHUNK // 16):
        ones_v[pl.ds(v * 16, 16)] = jnp.ones((16,), jnp.float32)
    plsc.subcore_barrier()

    def body(j, carry):
        pltpu.sync_copy(ones_v, deg_sh.at[idx_v.at[j]], add=True)
        return carry

    lax.fori_loop(0, DNCHUNK, body, 0)
    plsc.subcore_barrier()
    pltpu.sync_copy(deg_sh.at[pl.ds(sid * STRIPE, STRIPE)],
                    out_hbm.at[cid, 0, pl.ds(sid * STRIPE, STRIPE)])


_deg_kernel = functools.partial(
    pl.kernel,
    out_type=jax.ShapeDtypeStruct((NC, 1, NPAD), jnp.float32),
    mesh=_MESH(),
    scratch_types=[
        pltpu.VMEM((DNCHUNK, DCHUNK), jnp.int32),
        pltpu.VMEM((DCHUNK,), jnp.float32),
        pltpu.VMEM_SHARED((NPAD,), jnp.float32),
    ],
)(_deg_body)


def _spmm_body(y_hbm, col_hbm, row_hbm, zeros_hbm, out_hbm,
               idx_c, idx_r, buf0, buf1, acc_sh, sem0, sem1):
    bufs = (buf0, buf1)
    sems = (sem0, sem1)
    cid = lax.axis_index("c")
    sid = lax.axis_index("s")
    wid = sid * NC + cid
    pltpu.sync_copy(zeros_hbm, acc_sh.at[pl.ds(sid * STRIPE, STRIPE)])
    plsc.subcore_barrier()

    for s in range(NSUPER):
        pltpu.sync_copy(col_hbm.at[wid, s], idx_c)
        pltpu.sync_copy(row_hbm.at[wid, s], idx_r)
        pltpu.async_copy(y_hbm.at[idx_c.at[0]], bufs[0], sems[0])

        def group(g, carry):
            for b in range(NBUF):
                j = g * NBUF + b
                nxt = j + 1
                nb = (b + 1) % NBUF

                @pl.when(nxt < SUPER)
                def _():
                    pltpu.async_copy(y_hbm.at[idx_c.at[nxt]], bufs[nb],
                                     sems[nb])

                pltpu.make_async_copy(y_hbm.at[idx_c.at[j]], bufs[b],
                                      sems[b]).wait()
                pltpu.sync_copy(bufs[b], acc_sh.at[idx_r.at[j]], add=True)
            return carry

        lax.fori_loop(0, SUPER // NBUF, group, 0)
    plsc.subcore_barrier()
    pltpu.sync_copy(acc_sh.at[pl.ds(sid * STRIPE, STRIPE)],
                    out_hbm.at[cid, pl.ds(sid * STRIPE, STRIPE)])


_spmm_kernel = functools.partial(
    pl.kernel,
    out_type=jax.ShapeDtypeStruct((NC, NPAD, D), jnp.float32),
    mesh=_MESH(),
    scratch_types=[
        pltpu.VMEM((SUPER, CHUNK), jnp.int32),
        pltpu.VMEM((SUPER, CHUNK), jnp.int32),
        pltpu.VMEM((CHUNK, D), jnp.float32),
        pltpu.VMEM((CHUNK, D), jnp.float32),
        pltpu.VMEM_SHARED((NPAD, D), jnp.float32),
        pltpu.SemaphoreType.DMA,
        pltpu.SemaphoreType.DMA,
    ],
)(_spmm_body)


BR = 2000


def _scale_body(p0_ref, p1_ref, x_ref, y_ref, dis_ref):
    deg = p0_ref[...] + p1_ref[...]
    dis = jnp.where(deg > 0.0, lax.rsqrt(deg), 0.0)
    dis_ref[...] = dis
    y_ref[...] = dis * x_ref[...]


def _tc_scale(p0, p1, x):
    return pl.pallas_call(
        _scale_body,
        grid=(N // BR,),
        in_specs=[
            pl.BlockSpec((BR, 1), lambda i: (i, 0)),
            pl.BlockSpec((BR, 1), lambda i: (i, 0)),
            pl.BlockSpec((BR, D), lambda i: (i, 0)),
        ],
        out_specs=[
            pl.BlockSpec((BR, D), lambda i: (i, 0)),
            pl.BlockSpec((BR, 1), lambda i: (i, 0)),
        ],
        out_shape=[
            jax.ShapeDtypeStruct((N, D), jnp.float32),
            jax.ShapeDtypeStruct((N, 1), jnp.float32),
        ],
    )(p0, p1, x)


def _mid_body(dis_ref, q0_ref, q1_ref, tx1_ref, y2_ref):
    dis = dis_ref[...]
    tx1 = -dis * (q0_ref[0] + q1_ref[0])
    tx1_ref[...] = tx1
    y2_ref[...] = dis * tx1


def _tc_mid(dis, agg):
    return pl.pallas_call(
        _mid_body,
        grid=(N // BR,),
        in_specs=[
            pl.BlockSpec((BR, 1), lambda i: (i, 0)),
            pl.BlockSpec((1, BR, D), lambda i: (0, i, 0)),
            pl.BlockSpec((1, BR, D), lambda i: (1, i, 0)),
        ],
        out_specs=[
            pl.BlockSpec((BR, D), lambda i: (i, 0)),
            pl.BlockSpec((BR, D), lambda i: (i, 0)),
        ],
        out_shape=[
            jax.ShapeDtypeStruct((N, D), jnp.float32),
            jax.ShapeDtypeStruct((N, D), jnp.float32),
        ],
    )(dis, agg, agg)


def _final_body(x_ref, tx1_ref, dis_ref, q0_ref, q1_ref,
                wz_ref, wh_ref, bz_ref, bh_ref, h_ref):
    x = x_ref[...]
    tx1 = tx1_ref[...]
    tx2 = -2.0 * dis_ref[...] * (q0_ref[0] + q1_ref[0]) - x
    dot = functools.partial(
        jnp.dot, precision=lax.Precision.HIGHEST,
        preferred_element_type=jnp.float32)
    sz = (dot(x, wz_ref[0]) + dot(tx1, wz_ref[1]) + dot(tx2, wz_ref[2])
          + bz_ref[...])
    sh = (dot(x, wh_ref[0]) + dot(tx1, wh_ref[1]) + dot(tx2, wh_ref[2])
          + bh_ref[...])
    h_ref[...] = (1.0 - jax.nn.sigmoid(sz)) * jnp.tanh(sh)


def _tc_final(x, tx1, dis, agg, wz, wh, bz, bh):
    return pl.pallas_call(
        _final_body,
        grid=(N // BR,),
        in_specs=[
            pl.BlockSpec((BR, D), lambda i: (i, 0)),
            pl.BlockSpec((BR, D), lambda i: (i, 0)),
            pl.BlockSpec((BR, 1), lambda i: (i, 0)),
            pl.BlockSpec((1, BR, D), lambda i: (0, i, 0)),
            pl.BlockSpec((1, BR, D), lambda i: (1, i, 0)),
            pl.BlockSpec((3, D, D), lambda i: (0, 0, 0)),
            pl.BlockSpec((3, D, D), lambda i: (0, 0, 0)),
            pl.BlockSpec((1, D), lambda i: (0, 0)),
            pl.BlockSpec((1, D), lambda i: (0, 0)),
        ],
        out_specs=pl.BlockSpec((BR, D), lambda i: (i, 0)),
        out_shape=jax.ShapeDtypeStruct((N, D), jnp.float32),
    )(x, tx1, dis, agg, agg, wz, wh, bz, bh)


def kernel(X, edge_index, W_xz, b_xz, W_hz, b_hz, W_xr, b_xr, W_hr, b_hr,
           W_xh, b_xh, W_hh, b_hh):
    row4d = edge_index[0].reshape(NW, NSUPER, SUPER, CHUNK)
    col4d = edge_index[1].reshape(NW, NSUPER, SUPER, CHUNK)
    row3d = edge_index[0].reshape(NW, DNCHUNK, DCHUNK)
    z_row = jnp.zeros((STRIPE,), jnp.float32)
    z_acc = jnp.zeros((STRIPE, D), jnp.float32)

    deg_p = _deg_kernel(row3d, z_row)
    p0 = deg_p[0, 0, :N, None]
    p1 = deg_p[1, 0, :N, None]
    y1, dis = _tc_scale(p0, p1, X)

    agg1 = _spmm_kernel(y1, col4d, row4d, z_acc)
    tx1, y2 = _tc_mid(dis, agg1)

    agg2 = _spmm_kernel(y2, col4d, row4d, z_acc)
    bz = (b_xz + b_hz)[None, :]
    bh = (b_xh + b_hh)[None, :]
    return _tc_final(X, tx1, dis, agg2, W_xz, W_xh, bz, bh)
